# baseline (speedup 1.0000x reference)
import jax
import jax.numpy as jnp
from jax import lax
from jax.experimental import pallas as pl
from jax.experimental.pallas import tpu as pltpu
from jax.experimental.pallas import tpu_sc as plsc

N = 10000
T = 8
OBS = 128
H = 128
ACT = 8
E = 320000

AUG = H + 16
BN = 1000
GRID = N // BN

NC = 2
NS = 16
NW = NC * NS
EPW = E // NW
K = 16
NCH = EPW // K
NVR = AUG // 16

_f32 = jnp.float32
_i32 = jnp.int32


def _lstm_body(x_ref, wih_ref, whh_ref, b_ref, o_ref):
    x = x_ref[...]
    h = jnp.zeros((BN, H), _f32)
    c = jnp.zeros((BN, H), _f32)
    wih = wih_ref[...]
    whh = whh_ref[...]
    b = b_ref[...]
    for t in range(T):
        xt = x[:, t, :]
        z = (jnp.dot(xt, wih, preferred_element_type=_f32)
             + jnp.dot(h, whh, preferred_element_type=_f32) + b)
        i_ = jax.nn.sigmoid(z[:, 0:H])
        f_ = jax.nn.sigmoid(z[:, H:2 * H])
        g_ = jnp.tanh(z[:, 2 * H:3 * H])
        o_ = jax.nn.sigmoid(z[:, 3 * H:4 * H])
        c = f_ * c + i_ * g_
        h = o_ * jnp.tanh(c)
    o_ref[...] = h


def _lstm(x_seq, wih_t, whh_t, bias):
    return pl.pallas_call(
        _lstm_body,
        grid=(GRID,),
        in_specs=[
            pl.BlockSpec((BN, T, OBS), lambda i: (i, 0, 0)),
            pl.BlockSpec((OBS, 4 * H), lambda i: (0, 0)),
            pl.BlockSpec((H, 4 * H), lambda i: (0, 0)),
            pl.BlockSpec((1, 4 * H), lambda i: (0, 0)),
        ],
        out_specs=pl.BlockSpec((BN, H), lambda i: (i, 0)),
        out_shape=jax.ShapeDtypeStruct((N, H), _f32),
    )(x_seq, wih_t, whh_t, bias)


def _proj_body(x_ref, wl_ref, wr_ref, xla_ref, xr_ref):
    x = x_ref[...]
    xl = jnp.dot(x, wl_ref[...], preferred_element_type=_f32)
    ones = jnp.ones((BN, 1), _f32)
    zer = jnp.zeros((BN, AUG - H - 1), _f32)
    xla_ref[...] = jnp.concatenate([xl, ones, zer], axis=1)
    xr_ref[...] = jnp.dot(x, wr_ref[...], preferred_element_type=_f32)


def _proj(x, wl, wr):
    return pl.pallas_call(
        _proj_body,
        grid=(GRID,),
        in_specs=[
            pl.BlockSpec((BN, H), lambda i: (i, 0)),
            pl.BlockSpec((H, H), lambda i: (0, 0)),
            pl.BlockSpec((H, H), lambda i: (0, 0)),
        ],
        out_specs=[
            pl.BlockSpec((BN, AUG), lambda i: (i, 0)),
            pl.BlockSpec((BN, H), lambda i: (i, 0)),
        ],
        out_shape=[
            jax.ShapeDtypeStruct((N, AUG), _f32),
            jax.ShapeDtypeStruct((N, H), _f32),
        ],
    )(x, wl, wr)


def _hsum_splat(v):
    for sh in (8, 4, 2, 1):
        perm = jnp.bitwise_xor(lax.iota(_i32, 16), sh)
        v = v + jnp.take(v, perm)
    return v


NB = 5


def _gat_edge_kernel(*refs):
    (xla_hbm, xr_hbm, src_hbm, dst_hbm, ea_hbm, att_hbm, we_hbm,
     out_hbm) = refs[:8]
    i0 = 8
    srcb = list(refs[i0:i0 + NB]); i0 += NB
    dstb = list(refs[i0:i0 + NB]); i0 += NB
    eab = list(refs[i0:i0 + NB]); i0 += NB
    dstS = list(refs[i0:i0 + NB]); i0 += NB
    xlb = list(refs[i0:i0 + NB]); i0 += NB
    xrb = list(refs[i0:i0 + NB]); i0 += NB
    orb = list(refs[i0:i0 + NB]); i0 += NB
    attbuf, webuf, num_sh = refs[i0:i0 + 3]; i0 += 3
    isem = list(refs[i0:i0 + NB]); i0 += NB
    gsem = list(refs[i0:i0 + NB]); i0 += NB
    ssem = list(refs[i0:i0 + NB]); i0 += NB

    cid = lax.axis_index("c")
    sid = lax.axis_index("s")
    wid = sid * NC + cid
    ebase = wid * EPW

    pltpu.sync_copy(att_hbm, attbuf)
    pltpu.sync_copy(we_hbm, webuf)
    att_v = [attbuf[pl.ds(j * 16, 16)] for j in range(H // 16)]
    we_v = [webuf[pl.ds(j * 16, 16)] for j in range(H // 16)]

    def fire_idx(c, b):
        base = ebase + c * K
        pltpu.async_copy(src_hbm.at[pl.ds(base, K)], srcb[b], isem[b])
        pltpu.async_copy(dst_hbm.at[pl.ds(base, K)], dstb[b], isem[b])
        pltpu.async_copy(ea_hbm.at[pl.ds(base, K)], eab[b], isem[b])

    def wait_idx(b):
        pltpu.make_async_copy(src_hbm.at[pl.ds(0, K)], srcb[b], isem[b]).wait()
        pltpu.make_async_copy(dst_hbm.at[pl.ds(0, K)], dstb[b], isem[b]).wait()
        pltpu.make_async_copy(ea_hbm.at[pl.ds(0, K)], eab[b], isem[b]).wait()

    def fire_gathers(b):
        pltpu.async_copy(xla_hbm.at[srcb[b]], xlb[b], gsem[b])
        pltpu.async_copy(xr_hbm.at[dstb[b]], xrb[b], gsem[b])

    def wait_gathers(b):
        pltpu.make_async_copy(xla_hbm.at[srcb[b]], xlb[b], gsem[b]).wait()
        pltpu.make_async_copy(xr_hbm.at[dstb[b]], xrb[b], gsem[b]).wait()

    def fire_scatter(b):
        pltpu.async_copy(orb[b], num_sh.at[dstS[b]], ssem[b], add=True)

    def wait_scatter(b):
        pltpu.make_async_copy(orb[b], num_sh.at[dstS[b]], ssem[b]).wait()

    for b in range(NB):
        fire_idx(b, b)

    wait_idx(0)
    fire_gathers(0)
    wait_idx(1)
    fire_gathers(1)

    for i in range(K):
        for j in range(NVR):
            orb[0][i, pl.ds(j * 16, 16)] = jnp.zeros((16,), _f32)
    nb_tot = N // K
    nblk = (nb_tot - 1 - sid) // NS + 1

    def zfire(t, _):
        rb = (sid + t * NS) * K
        pltpu.async_copy(orb[0], num_sh.at[pl.ds(rb, K)], ssem[0])
        return 0
    lax.fori_loop(0, nblk, zfire, 0)

    def zdrain(t, _):
        pltpu.make_async_copy(orb[0], num_sh.at[pl.ds(0, K)], ssem[0]).wait()
        return 0
    lax.fori_loop(0, nblk, zdrain, 0)
    plsc.subcore_barrier()

    def compute_inplace(b):
        eavec = eab[b][...]

        @plsc.parallel_loop(0, K, unroll=2)
        def _edge(e):
            eas = jnp.take(eavec, jnp.full((16,), e, _i32))
            acc0 = jnp.zeros((16,), _f32)
            acc1 = jnp.zeros((16,), _f32)
            xlv = []
            for j in range(H // 16):
                sl = pl.ds(j * 16, 16)
                xl_j = xlb[b][e, sl]
                xr_j = xrb[b][e, sl]
                xlv.append(xl_j)
                u = xl_j + xr_j + eas * we_v[j]
                t = att_v[j] * jnp.maximum(u, 0.2 * u)
                if j % 2 == 0:
                    acc0 = acc0 + t
                else:
                    acc1 = acc1 + t
            w = jnp.exp(_hsum_splat(acc0 + acc1))
            for j in range(H // 16):
                orb[b][e, pl.ds(j * 16, 16)] = xlv[j] * w
            sl = pl.ds(H, 16)
            orb[b][e, sl] = xlb[b][e, sl] * w

    def site(i, b):
        c = NB * i + b
        b2 = (b + 2) % NB
        wait_gathers(b)
        compute_inplace(b)
        dstS[b][...] = dstb[b][...]
        fire_scatter(b)

        @pl.when(c + NB < NCH)
        def _():
            fire_idx(c + NB, b)

        @pl.when(c + 2 < NCH)
        def _():
            @pl.when(c >= 3)
            def _():
                wait_scatter(b2)
            wait_idx(b2)
            fire_gathers(b2)

    def iter_body(i, _):
        for b in range(NB):
            site(i, b)
        return 0
    lax.fori_loop(0, NCH // NB, iter_body, 0)

    for b in range(NB):
        wait_scatter(b)
    plsc.subcore_barrier()

    def wfire(t, _):
        rb = (sid + t * NS) * K
        pltpu.async_copy(num_sh.at[pl.ds(rb, K)], out_hbm.at[cid, pl.ds(rb, K)],
                         ssem[0])
        return 0
    lax.fori_loop(0, nblk, wfire, 0)

    def wdrain(t, _):
        pltpu.make_async_copy(num_sh.at[pl.ds(0, K)],
                              out_hbm.at[cid, pl.ds(0, K)], ssem[0]).wait()
        return 0
    lax.fori_loop(0, nblk, wdrain, 0)


def _gat_edge(xla, xr, src, dst, ea, att, we):
    mesh = plsc.VectorSubcoreMesh(core_axis_name="c", subcore_axis_name="s",
                                  num_cores=NC, num_subcores=NS)
    scratch = ([pltpu.VMEM((K,), _i32) for _ in range(NB)]
               + [pltpu.VMEM((K,), _i32) for _ in range(NB)]
               + [pltpu.VMEM((K,), _f32) for _ in range(NB)]
               + [pltpu.VMEM((K,), _i32) for _ in range(NB)]
               + [pltpu.VMEM((K, AUG), _f32) for _ in range(NB)]
               + [pltpu.VMEM((K, H), _f32) for _ in range(NB)]
               + [pltpu.VMEM((K, AUG), _f32) for _ in range(NB)]
               + [pltpu.VMEM((H,), _f32), pltpu.VMEM((H,), _f32)]
               + [pltpu.VMEM_SHARED((N, AUG), _f32)]
               + [pltpu.SemaphoreType.DMA for _ in range(3 * NB)])
    f = pl.kernel(
        _gat_edge_kernel,
        out_type=jax.ShapeDtypeStruct((NC, N, AUG), _f32),
        mesh=mesh,
        compiler_params=pltpu.CompilerParams(use_tc_tiling_on_sc=False),
        scratch_types=scratch,
    )
    return f(xla, xr, src, dst, ea, att, we)


def _combine_proj_body(p_ref, b_ref, wl_ref, wr_ref, x1_ref, xla_ref, xr_ref):
    p = p_ref[...]
    ps = p[0] + p[1]
    den = ps[:, H:H + 1]
    x1 = jnp.maximum(ps[:, :H] / (den + 1e-16) + b_ref[...], 0.0)
    x1_ref[...] = x1
    xl = jnp.dot(x1, wl_ref[...], preferred_element_type=_f32)
    ones = jnp.ones((BN, 1), _f32)
    zer = jnp.zeros((BN, AUG - H - 1), _f32)
    xla_ref[...] = jnp.concatenate([xl, ones, zer], axis=1)
    xr_ref[...] = jnp.dot(x1, wr_ref[...], preferred_element_type=_f32)


def _combine_proj(part, b, wl, wr):
    return pl.pallas_call(
        _combine_proj_body,
        grid=(GRID,),
        in_specs=[
            pl.BlockSpec((NC, BN, AUG), lambda i: (0, i, 0)),
            pl.BlockSpec((1, H), lambda i: (0, 0)),
            pl.BlockSpec((H, H), lambda i: (0, 0)),
            pl.BlockSpec((H, H), lambda i: (0, 0)),
        ],
        out_specs=[
            pl.BlockSpec((BN, H), lambda i: (i, 0)),
            pl.BlockSpec((BN, AUG), lambda i: (i, 0)),
            pl.BlockSpec((BN, H), lambda i: (i, 0)),
        ],
        out_shape=[
            jax.ShapeDtypeStruct((N, H), _f32),
            jax.ShapeDtypeStruct((N, AUG), _f32),
            jax.ShapeDtypeStruct((N, H), _f32),
        ],
    )(part, b, wl, wr)


def _final_body(p_ref, b_ref, x1_ref, a1w_ref, a1b_ref, a2w_ref, a2b_ref,
                c1w_ref, c1b_ref, c2w_ref, c2b_ref, ls_ref,
                mu_ref, std_ref, val_ref):
    p = p_ref[...]
    ps = p[0] + p[1]
    den = ps[:, H:H + 1]
    x2 = (jnp.maximum(ps[:, :H] / (den + 1e-16) + b_ref[...], 0.0)
          + x1_ref[...])
    ha = jnp.maximum(
        jnp.dot(x2, a1w_ref[...], preferred_element_type=_f32) + a1b_ref[...],
        0.0)
    mu_ref[...] = (jnp.dot(ha, a2w_ref[...], preferred_element_type=_f32)
                   + a2b_ref[...])
    std_ref[...] = jnp.broadcast_to(jnp.exp(ls_ref[...]), (BN, ACT))
    hv = jnp.maximum(
        jnp.dot(x2, c1w_ref[...], preferred_element_type=_f32) + c1b_ref[...],
        0.0)
    val_ref[...] = (jnp.dot(hv, c2w_ref[...], preferred_element_type=_f32)
                    + c2b_ref[...])


def _final(part, b, x1, a1w, a1b, a2w, a2b, c1w, c1b, c2w, c2b, log_std):
    return pl.pallas_call(
        _final_body,
        grid=(GRID,),
        in_specs=[
            pl.BlockSpec((NC, BN, AUG), lambda i: (0, i, 0)),
            pl.BlockSpec((1, H), lambda i: (0, 0)),
            pl.BlockSpec((BN, H), lambda i: (i, 0)),
            pl.BlockSpec((H, H), lambda i: (0, 0)),
            pl.BlockSpec((1, H), lambda i: (0, 0)),
            pl.BlockSpec((H, ACT), lambda i: (0, 0)),
            pl.BlockSpec((1, ACT), lambda i: (0, 0)),
            pl.BlockSpec((H, H), lambda i: (0, 0)),
            pl.BlockSpec((1, H), lambda i: (0, 0)),
            pl.BlockSpec((H, 1), lambda i: (0, 0)),
            pl.BlockSpec((1, 1), lambda i: (0, 0)),
            pl.BlockSpec((1, ACT), lambda i: (0, 0)),
        ],
        out_specs=[
            pl.BlockSpec((BN, ACT), lambda i: (i, 0)),
            pl.BlockSpec((BN, ACT), lambda i: (i, 0)),
            pl.BlockSpec((BN, 1), lambda i: (i, 0)),
        ],
        out_shape=[
            jax.ShapeDtypeStruct((N, ACT), _f32),
            jax.ShapeDtypeStruct((N, ACT), _f32),
            jax.ShapeDtypeStruct((N, 1), _f32),
        ],
    )(part, b, x1, a1w, a1b, a2w, a2b, c1w, c1b, c2w, c2b, log_std)


def kernel(x_seq, edge_index, edge_attr, Wih, Whh, bih, bhh, Wl1, Wr1, We1,
           att1, b1, Wl2, Wr2, We2, att2, b2, A1w, A1b, A2w, A2b, C1w, C1b,
           C2w, C2b, log_std):
    src = edge_index[0]
    dst = edge_index[1]
    ea = edge_attr[:, 0]

    final = _lstm(x_seq, Wih.T, Whh.T, (bih + bhh).reshape(1, 4 * H))

    xla1, xr1 = _proj(final, Wl1, Wr1)
    part1 = _gat_edge(xla1, xr1, src, dst, ea, att1, We1[0])
    x1, xla2, xr2 = _combine_proj(part1, b1.reshape(1, H), Wl2, Wr2)
    part2 = _gat_edge(xla2, xr2, src, dst, ea, att2, We2[0])
    mu, std, value = _final(part2, b2.reshape(1, H), x1, A1w,
                            A1b.reshape(1, H), A2w, A2b.reshape(1, ACT), C1w,
                            C1b.reshape(1, H), C2w, C2b.reshape(1, 1),
                            log_std.reshape(1, ACT))
    return (mu, std, value[:, 0])


# speedup vs baseline: 1.0905x; 1.0905x over previous
import jax
import jax.numpy as jnp
from jax import lax
from jax.experimental import pallas as pl
from jax.experimental.pallas import tpu as pltpu
from jax.experimental.pallas import tpu_sc as plsc

N = 10000
T = 8
OBS = 128
H = 128
ACT = 8
E = 320000

AUG = H + 16
BN = 1000
GRID = N // BN

NC = 2
NS = 16
NW = NC * NS
EPW = E // NW
K = 16
NCH = EPW // K
NVR = AUG // 16

_f32 = jnp.float32
_i32 = jnp.int32


def _lstm_body(x_ref, wih_ref, whh_ref, b_ref, o_ref):
    x = x_ref[...].astype(jnp.bfloat16)
    h = jnp.zeros((BN, H), _f32)
    c = jnp.zeros((BN, H), _f32)
    wih = wih_ref[...].astype(jnp.bfloat16)
    whh = whh_ref[...].astype(jnp.bfloat16)
    b = b_ref[...]
    for t in range(T):
        xt = x[:, t, :]
        z = (jnp.dot(xt, wih, preferred_element_type=_f32)
             + jnp.dot(h.astype(jnp.bfloat16), whh,
                       preferred_element_type=_f32) + b)
        i_ = jax.nn.sigmoid(z[:, 0:H])
        f_ = jax.nn.sigmoid(z[:, H:2 * H])
        g_ = jnp.tanh(z[:, 2 * H:3 * H])
        o_ = jax.nn.sigmoid(z[:, 3 * H:4 * H])
        c = f_ * c + i_ * g_
        h = o_ * jnp.tanh(c)
    o_ref[...] = h


def _lstm(x_seq, wih_t, whh_t, bias):
    return pl.pallas_call(
        _lstm_body,
        grid=(GRID,),
        in_specs=[
            pl.BlockSpec((BN, T, OBS), lambda i: (i, 0, 0)),
            pl.BlockSpec((OBS, 4 * H), lambda i: (0, 0)),
            pl.BlockSpec((H, 4 * H), lambda i: (0, 0)),
            pl.BlockSpec((1, 4 * H), lambda i: (0, 0)),
        ],
        out_specs=pl.BlockSpec((BN, H), lambda i: (i, 0)),
        out_shape=jax.ShapeDtypeStruct((N, H), _f32),
    )(x_seq, wih_t, whh_t, bias)


def _proj_body(x_ref, wl_ref, wr_ref, xla_ref, xr_ref):
    x = x_ref[...]
    xl = jnp.dot(x, wl_ref[...], preferred_element_type=_f32)
    ones = jnp.ones((BN, 1), _f32)
    zer = jnp.zeros((BN, AUG - H - 1), _f32)
    xla_ref[...] = jnp.concatenate([xl, ones, zer], axis=1)
    xr_ref[...] = jnp.dot(x, wr_ref[...], preferred_element_type=_f32)


def _proj(x, wl, wr):
    return pl.pallas_call(
        _proj_body,
        grid=(GRID,),
        in_specs=[
            pl.BlockSpec((BN, H), lambda i: (i, 0)),
            pl.BlockSpec((H, H), lambda i: (0, 0)),
            pl.BlockSpec((H, H), lambda i: (0, 0)),
        ],
        out_specs=[
            pl.BlockSpec((BN, AUG), lambda i: (i, 0)),
            pl.BlockSpec((BN, H), lambda i: (i, 0)),
        ],
        out_shape=[
            jax.ShapeDtypeStruct((N, AUG), _f32),
            jax.ShapeDtypeStruct((N, H), _f32),
        ],
    )(x, wl, wr)


def _hsum_splat(v):
    for sh in (8, 4, 2, 1):
        perm = jnp.bitwise_xor(lax.iota(_i32, 16), sh)
        v = v + jnp.take(v, perm)
    return v


NB = 5


def _gat_edge_kernel(*refs):
    (xla_hbm, xr_hbm, src_hbm, dst_hbm, ea_hbm, att_hbm, we_hbm,
     out_hbm) = refs[:8]
    i0 = 8
    srcb = list(refs[i0:i0 + NB]); i0 += NB
    dstb = list(refs[i0:i0 + NB]); i0 += NB
    eab = list(refs[i0:i0 + NB]); i0 += NB
    dstS = list(refs[i0:i0 + NB]); i0 += NB
    xlb = list(refs[i0:i0 + NB]); i0 += NB
    xrb = list(refs[i0:i0 + NB]); i0 += NB
    attbuf, webuf, num_sh = refs[i0:i0 + 3]; i0 += 3
    isem = list(refs[i0:i0 + NB]); i0 += NB
    gsem = list(refs[i0:i0 + NB]); i0 += NB
    ssem = list(refs[i0:i0 + NB]); i0 += NB

    cid = lax.axis_index("c")
    sid = lax.axis_index("s")
    wid = sid * NC + cid
    ebase = wid * EPW

    pltpu.sync_copy(att_hbm, attbuf)
    pltpu.sync_copy(we_hbm, webuf)
    att_v = [attbuf[pl.ds(j * 16, 16)] for j in range(H // 16)]
    we_v = [webuf[pl.ds(j * 16, 16)] for j in range(H // 16)]

    def fire_idx(c, b):
        base = ebase + c * K
        pltpu.async_copy(src_hbm.at[pl.ds(base, K)], srcb[b], isem[b])
        pltpu.async_copy(dst_hbm.at[pl.ds(base, K)], dstb[b], isem[b])
        pltpu.async_copy(ea_hbm.at[pl.ds(base, K)], eab[b], isem[b])

    def wait_idx(b):
        pltpu.make_async_copy(src_hbm.at[pl.ds(0, K)], srcb[b], isem[b]).wait()
        pltpu.make_async_copy(dst_hbm.at[pl.ds(0, K)], dstb[b], isem[b]).wait()
        pltpu.make_async_copy(ea_hbm.at[pl.ds(0, K)], eab[b], isem[b]).wait()

    def fire_gathers(b):
        pltpu.async_copy(xla_hbm.at[srcb[b]], xlb[b], gsem[b])
        pltpu.async_copy(xr_hbm.at[dstb[b]], xrb[b], gsem[b])

    def wait_gathers(b):
        pltpu.make_async_copy(xla_hbm.at[srcb[b]], xlb[b], gsem[b]).wait()
        pltpu.make_async_copy(xr_hbm.at[dstb[b]], xrb[b], gsem[b]).wait()

    def fire_scatter(b):
        pltpu.async_copy(xlb[b], num_sh.at[dstS[b]], ssem[b], add=True)

    def wait_scatter(b):
        pltpu.make_async_copy(xlb[b], num_sh.at[dstS[b]], ssem[b]).wait()

    for b in range(NB):
        fire_idx(b, b)

    wait_idx(0)
    fire_gathers(0)
    wait_idx(1)
    fire_gathers(1)

    for i in range(K):
        for j in range(NVR):
            xlb[2][i, pl.ds(j * 16, 16)] = jnp.zeros((16,), _f32)
    nb_tot = N // K
    nblk = (nb_tot - 1 - sid) // NS + 1

    def zfire(t, _):
        rb = (sid + t * NS) * K
        pltpu.async_copy(xlb[2], num_sh.at[pl.ds(rb, K)], ssem[0])
        return 0
    lax.fori_loop(0, nblk, zfire, 0)

    def zdrain(t, _):
        pltpu.make_async_copy(xlb[2], num_sh.at[pl.ds(0, K)], ssem[0]).wait()
        return 0
    lax.fori_loop(0, nblk, zdrain, 0)
    plsc.subcore_barrier()

    def compute_inplace(b):
        eavec = eab[b][...]

        @plsc.parallel_loop(0, K, unroll=2)
        def _edge(e):
            eas = jnp.take(eavec, jnp.full((16,), e, _i32))
            acc0 = jnp.zeros((16,), _f32)
            acc1 = jnp.zeros((16,), _f32)
            xlv = []
            for j in range(H // 16):
                sl = pl.ds(j * 16, 16)
                xl_j = xlb[b][e, sl]
                xr_j = xrb[b][e, sl]
                xlv.append(xl_j)
                u = xl_j + xr_j + eas * we_v[j]
                t = att_v[j] * jnp.maximum(u, 0.2 * u)
                if j % 2 == 0:
                    acc0 = acc0 + t
                else:
                    acc1 = acc1 + t
            w = jnp.exp(_hsum_splat(acc0 + acc1))
            for j in range(H // 16):
                xlb[b][e, pl.ds(j * 16, 16)] = xlv[j] * w
            sl = pl.ds(H, 16)
            xlb[b][e, sl] = xlb[b][e, sl] * w

    def site(i, b):
        c = NB * i + b
        b2 = (b + 2) % NB
        wait_gathers(b)
        compute_inplace(b)
        dstS[b][...] = dstb[b][...]
        fire_scatter(b)

        @pl.when(c + NB < NCH)
        def _():
            fire_idx(c + NB, b)

        @pl.when(c + 2 < NCH)
        def _():
            @pl.when(c >= 3)
            def _():
                wait_scatter(b2)
            wait_idx(b2)
            fire_gathers(b2)

    def iter_body(i, _):
        for b in range(NB):
            site(i, b)
        return 0
    lax.fori_loop(0, NCH // NB, iter_body, 0)

    for b in range(NB):
        wait_scatter(b)
    plsc.subcore_barrier()

    def wfire(t, _):
        rb = (sid + t * NS) * K
        pltpu.async_copy(num_sh.at[pl.ds(rb, K)], out_hbm.at[cid, pl.ds(rb, K)],
                         ssem[0])
        return 0
    lax.fori_loop(0, nblk, wfire, 0)

    def wdrain(t, _):
        pltpu.make_async_copy(num_sh.at[pl.ds(0, K)],
                              out_hbm.at[cid, pl.ds(0, K)], ssem[0]).wait()
        return 0
    lax.fori_loop(0, nblk, wdrain, 0)


def _gat_edge(xla, xr, src, dst, ea, att, we):
    mesh = plsc.VectorSubcoreMesh(core_axis_name="c", subcore_axis_name="s",
                                  num_cores=NC, num_subcores=NS)
    scratch = ([pltpu.VMEM((K,), _i32) for _ in range(NB)]
               + [pltpu.VMEM((K,), _i32) for _ in range(NB)]
               + [pltpu.VMEM((K,), _f32) for _ in range(NB)]
               + [pltpu.VMEM((K,), _i32) for _ in range(NB)]
               + [pltpu.VMEM((K, AUG), _f32) for _ in range(NB)]
               + [pltpu.VMEM((K, H), _f32) for _ in range(NB)]
               + [pltpu.VMEM((H,), _f32), pltpu.VMEM((H,), _f32)]
               + [pltpu.VMEM_SHARED((N, AUG), _f32)]
               + [pltpu.SemaphoreType.DMA for _ in range(3 * NB)])
    f = pl.kernel(
        _gat_edge_kernel,
        out_type=jax.ShapeDtypeStruct((NC, N, AUG), _f32),
        mesh=mesh,
        compiler_params=pltpu.CompilerParams(use_tc_tiling_on_sc=False),
        scratch_types=scratch,
    )
    return f(xla, xr, src, dst, ea, att, we)


def _combine_proj_body(p_ref, b_ref, wl_ref, wr_ref, x1_ref, xla_ref, xr_ref):
    p = p_ref[...]
    ps = p[0] + p[1]
    den = ps[:, H:H + 1]
    x1 = jnp.maximum(ps[:, :H] / (den + 1e-16) + b_ref[...], 0.0)
    x1_ref[...] = x1
    xl = jnp.dot(x1, wl_ref[...], preferred_element_type=_f32)
    ones = jnp.ones((BN, 1), _f32)
    zer = jnp.zeros((BN, AUG - H - 1), _f32)
    xla_ref[...] = jnp.concatenate([xl, ones, zer], axis=1)
    xr_ref[...] = jnp.dot(x1, wr_ref[...], preferred_element_type=_f32)


def _combine_proj(part, b, wl, wr):
    return pl.pallas_call(
        _combine_proj_body,
        grid=(GRID,),
        in_specs=[
            pl.BlockSpec((NC, BN, AUG), lambda i: (0, i, 0)),
            pl.BlockSpec((1, H), lambda i: (0, 0)),
            pl.BlockSpec((H, H), lambda i: (0, 0)),
            pl.BlockSpec((H, H), lambda i: (0, 0)),
        ],
        out_specs=[
            pl.BlockSpec((BN, H), lambda i: (i, 0)),
            pl.BlockSpec((BN, AUG), lambda i: (i, 0)),
            pl.BlockSpec((BN, H), lambda i: (i, 0)),
        ],
        out_shape=[
            jax.ShapeDtypeStruct((N, H), _f32),
            jax.ShapeDtypeStruct((N, AUG), _f32),
            jax.ShapeDtypeStruct((N, H), _f32),
        ],
    )(part, b, wl, wr)


def _final_body(p_ref, b_ref, x1_ref, a1w_ref, a1b_ref, a2w_ref, a2b_ref,
                c1w_ref, c1b_ref, c2w_ref, c2b_ref, ls_ref,
                mu_ref, std_ref, val_ref):
    p = p_ref[...]
    ps = p[0] + p[1]
    den = ps[:, H:H + 1]
    x2 = (jnp.maximum(ps[:, :H] / (den + 1e-16) + b_ref[...], 0.0)
          + x1_ref[...])
    ha = jnp.maximum(
        jnp.dot(x2, a1w_ref[...], preferred_element_type=_f32) + a1b_ref[...],
        0.0)
    mu_ref[...] = (jnp.dot(ha, a2w_ref[...], preferred_element_type=_f32)
                   + a2b_ref[...])
    std_ref[...] = jnp.broadcast_to(jnp.exp(ls_ref[...]), (BN, ACT))
    hv = jnp.maximum(
        jnp.dot(x2, c1w_ref[...], preferred_element_type=_f32) + c1b_ref[...],
        0.0)
    val_ref[...] = (jnp.dot(hv, c2w_ref[...], preferred_element_type=_f32)
                    + c2b_ref[...])


def _final(part, b, x1, a1w, a1b, a2w, a2b, c1w, c1b, c2w, c2b, log_std):
    return pl.pallas_call(
        _final_body,
        grid=(GRID,),
        in_specs=[
            pl.BlockSpec((NC, BN, AUG), lambda i: (0, i, 0)),
            pl.BlockSpec((1, H), lambda i: (0, 0)),
            pl.BlockSpec((BN, H), lambda i: (i, 0)),
            pl.BlockSpec((H, H), lambda i: (0, 0)),
            pl.BlockSpec((1, H), lambda i: (0, 0)),
            pl.BlockSpec((H, ACT), lambda i: (0, 0)),
            pl.BlockSpec((1, ACT), lambda i: (0, 0)),
            pl.BlockSpec((H, H), lambda i: (0, 0)),
            pl.BlockSpec((1, H), lambda i: (0, 0)),
            pl.BlockSpec((H, 1), lambda i: (0, 0)),
            pl.BlockSpec((1, 1), lambda i: (0, 0)),
            pl.BlockSpec((1, ACT), lambda i: (0, 0)),
        ],
        out_specs=[
            pl.BlockSpec((BN, ACT), lambda i: (i, 0)),
            pl.BlockSpec((BN, ACT), lambda i: (i, 0)),
            pl.BlockSpec((BN, 1), lambda i: (i, 0)),
        ],
        out_shape=[
            jax.ShapeDtypeStruct((N, ACT), _f32),
            jax.ShapeDtypeStruct((N, ACT), _f32),
            jax.ShapeDtypeStruct((N, 1), _f32),
        ],
    )(part, b, x1, a1w, a1b, a2w, a2b, c1w, c1b, c2w, c2b, log_std)


def kernel(x_seq, edge_index, edge_attr, Wih, Whh, bih, bhh, Wl1, Wr1, We1,
           att1, b1, Wl2, Wr2, We2, att2, b2, A1w, A1b, A2w, A2b, C1w, C1b,
           C2w, C2b, log_std):
    src = edge_index[0]
    dst = edge_index[1]
    ea = edge_attr[:, 0]

    final = _lstm(x_seq, Wih.T, Whh.T, (bih + bhh).reshape(1, 4 * H))

    xla1, xr1 = _proj(final, Wl1, Wr1)
    part1 = _gat_edge(xla1, xr1, src, dst, ea, att1, We1[0])
    x1, xla2, xr2 = _combine_proj(part1, b1.reshape(1, H), Wl2, Wr2)
    part2 = _gat_edge(xla2, xr2, src, dst, ea, att2, We2[0])
    mu, std, value = _final(part2, b2.reshape(1, H), x1, A1w,
                            A1b.reshape(1, H), A2w, A2b.reshape(1, ACT), C1w,
                            C1b.reshape(1, H), C2w, C2b.reshape(1, 1),
                            log_std.reshape(1, ACT))
    return (mu, std, value[:, 0])


# speedup vs baseline: 1.1011x; 1.0098x over previous
import jax
import jax.numpy as jnp
from jax import lax
from jax.experimental import pallas as pl
from jax.experimental.pallas import tpu as pltpu
from jax.experimental.pallas import tpu_sc as plsc

N = 10000
T = 8
OBS = 128
H = 128
ACT = 8
E = 320000

AUG = H + 16
BN = 1000
GRID = N // BN

NC = 2
NS = 16
NW = NC * NS
EPW = E // NW
K = 16
NCH = EPW // K
NVR = AUG // 16

_f32 = jnp.float32
_i32 = jnp.int32


def _lstm_body(x_ref, wih_ref, whh_ref, b_ref, wl_ref, wr_ref,
               xla_ref, xr_ref):
    x = x_ref[...]
    h = jnp.zeros((BN, H), _f32)
    c = jnp.zeros((BN, H), _f32)
    wih = wih_ref[...]
    whh = whh_ref[...]
    b = b_ref[...]
    for t in range(T):
        xt = x[:, t, :]
        z = (jnp.dot(xt, wih, preferred_element_type=_f32)
             + jnp.dot(h, whh, preferred_element_type=_f32) + b)
        i_ = jax.nn.sigmoid(z[:, 0:H])
        f_ = jax.nn.sigmoid(z[:, H:2 * H])
        g_ = jnp.tanh(z[:, 2 * H:3 * H])
        o_ = jax.nn.sigmoid(z[:, 3 * H:4 * H])
        c = f_ * c + i_ * g_
        h = o_ * jnp.tanh(c)
    xl = jnp.dot(h, wl_ref[...], preferred_element_type=_f32)
    ones = jnp.ones((BN, 1), _f32)
    zer = jnp.zeros((BN, AUG - H - 1), _f32)
    xla_ref[...] = jnp.concatenate([xl, ones, zer], axis=1)
    xr_ref[...] = jnp.dot(h, wr_ref[...], preferred_element_type=_f32)


def _lstm_proj(x_seq, wih_t, whh_t, bias, wl, wr):
    return pl.pallas_call(
        _lstm_body,
        grid=(GRID,),
        in_specs=[
            pl.BlockSpec((BN, T, OBS), lambda i: (i, 0, 0)),
            pl.BlockSpec((OBS, 4 * H), lambda i: (0, 0)),
            pl.BlockSpec((H, 4 * H), lambda i: (0, 0)),
            pl.BlockSpec((1, 4 * H), lambda i: (0, 0)),
            pl.BlockSpec((H, H), lambda i: (0, 0)),
            pl.BlockSpec((H, H), lambda i: (0, 0)),
        ],
        out_specs=[
            pl.BlockSpec((BN, AUG), lambda i: (i, 0)),
            pl.BlockSpec((BN, H), lambda i: (i, 0)),
        ],
        out_shape=[
            jax.ShapeDtypeStruct((N, AUG), _f32),
            jax.ShapeDtypeStruct((N, H), _f32),
        ],
    )(x_seq, wih_t, whh_t, bias, wl, wr)


def _hsum_splat(v):
    for sh in (8, 4, 2, 1):
        perm = jnp.bitwise_xor(lax.iota(_i32, 16), sh)
        v = v + jnp.take(v, perm)
    return v


NB = 5


def _gat_edge_kernel(*refs):
    (xla_hbm, xr_hbm, src_hbm, dst_hbm, ea_hbm, att_hbm, we_hbm,
     out_hbm) = refs[:8]
    i0 = 8
    srcb = list(refs[i0:i0 + NB]); i0 += NB
    dstb = list(refs[i0:i0 + NB]); i0 += NB
    eab = list(refs[i0:i0 + NB]); i0 += NB
    dstS = list(refs[i0:i0 + NB]); i0 += NB
    xlb = list(refs[i0:i0 + NB]); i0 += NB
    xrb = list(refs[i0:i0 + NB]); i0 += NB
    attbuf, webuf, num_sh = refs[i0:i0 + 3]; i0 += 3
    isem = list(refs[i0:i0 + NB]); i0 += NB
    gsem = list(refs[i0:i0 + NB]); i0 += NB
    ssem = list(refs[i0:i0 + NB]); i0 += NB

    cid = lax.axis_index("c")
    sid = lax.axis_index("s")
    wid = sid * NC + cid
    ebase = wid * EPW

    pltpu.sync_copy(att_hbm, attbuf)
    pltpu.sync_copy(we_hbm, webuf)
    att_v = [attbuf[pl.ds(j * 16, 16)] for j in range(H // 16)]
    we_v = [webuf[pl.ds(j * 16, 16)] for j in range(H // 16)]

    def fire_idx(c, b):
        base = ebase + c * K
        pltpu.async_copy(src_hbm.at[pl.ds(base, K)], srcb[b], isem[b])
        pltpu.async_copy(dst_hbm.at[pl.ds(base, K)], dstb[b], isem[b])
        pltpu.async_copy(ea_hbm.at[pl.ds(base, K)], eab[b], isem[b])

    def wait_idx(b):
        pltpu.make_async_copy(src_hbm.at[pl.ds(0, K)], srcb[b], isem[b]).wait()
        pltpu.make_async_copy(dst_hbm.at[pl.ds(0, K)], dstb[b], isem[b]).wait()
        pltpu.make_async_copy(ea_hbm.at[pl.ds(0, K)], eab[b], isem[b]).wait()

    def fire_gathers(b):
        pltpu.async_copy(xla_hbm.at[srcb[b]], xlb[b], gsem[b])
        pltpu.async_copy(xr_hbm.at[dstb[b]], xrb[b], gsem[b])

    def wait_gathers(b):
        pltpu.make_async_copy(xla_hbm.at[srcb[b]], xlb[b], gsem[b]).wait()
        pltpu.make_async_copy(xr_hbm.at[dstb[b]], xrb[b], gsem[b]).wait()

    def fire_scatter(b):
        pltpu.async_copy(xlb[b], num_sh.at[dstS[b]], ssem[b], add=True)

    def wait_scatter(b):
        pltpu.make_async_copy(xlb[b], num_sh.at[dstS[b]], ssem[b]).wait()

    for b in range(NB):
        fire_idx(b, b)

    wait_idx(0)
    fire_gathers(0)
    wait_idx(1)
    fire_gathers(1)

    for i in range(K):
        for j in range(NVR):
            xlb[2][i, pl.ds(j * 16, 16)] = jnp.zeros((16,), _f32)
    nb_tot = N // K
    nblk = (nb_tot - 1 - sid) // NS + 1

    def zfire(t, _):
        rb = (sid + t * NS) * K
        pltpu.async_copy(xlb[2], num_sh.at[pl.ds(rb, K)], ssem[0])
        return 0
    lax.fori_loop(0, nblk, zfire, 0)

    def zdrain(t, _):
        pltpu.make_async_copy(xlb[2], num_sh.at[pl.ds(0, K)], ssem[0]).wait()
        return 0
    lax.fori_loop(0, nblk, zdrain, 0)
    plsc.subcore_barrier()

    def compute_inplace(b):
        eavec = eab[b][...]

        @plsc.parallel_loop(0, K, unroll=2)
        def _edge(e):
            eas = jnp.take(eavec, jnp.full((16,), e, _i32))
            acc0 = jnp.zeros((16,), _f32)
            acc1 = jnp.zeros((16,), _f32)
            xlv = []
            for j in range(H // 16):
                sl = pl.ds(j * 16, 16)
                xl_j = xlb[b][e, sl]
                xr_j = xrb[b][e, sl]
                xlv.append(xl_j)
                u = xl_j + xr_j + eas * we_v[j]
                t = att_v[j] * jnp.maximum(u, 0.2 * u)
                if j % 2 == 0:
                    acc0 = acc0 + t
                else:
                    acc1 = acc1 + t
            w = jnp.exp(_hsum_splat(acc0 + acc1))
            for j in range(H // 16):
                xlb[b][e, pl.ds(j * 16, 16)] = xlv[j] * w
            sl = pl.ds(H, 16)
            xlb[b][e, sl] = xlb[b][e, sl] * w

    def site(i, b):
        c = NB * i + b
        b2 = (b + 2) % NB
        wait_gathers(b)
        compute_inplace(b)
        dstS[b][...] = dstb[b][...]
        fire_scatter(b)

        @pl.when(c + NB < NCH)
        def _():
            fire_idx(c + NB, b)

        @pl.when(c + 2 < NCH)
        def _():
            @pl.when(c >= 3)
            def _():
                wait_scatter(b2)
            wait_idx(b2)
            fire_gathers(b2)

    def iter_body(i, _):
        for b in range(NB):
            site(i, b)
        return 0
    lax.fori_loop(0, NCH // NB, iter_body, 0)

    for b in range(NB):
        wait_scatter(b)
    plsc.subcore_barrier()

    def wfire(t, _):
        rb = (sid + t * NS) * K
        pltpu.async_copy(num_sh.at[pl.ds(rb, K)], out_hbm.at[cid, pl.ds(rb, K)],
                         ssem[0])
        return 0
    lax.fori_loop(0, nblk, wfire, 0)

    def wdrain(t, _):
        pltpu.make_async_copy(num_sh.at[pl.ds(0, K)],
                              out_hbm.at[cid, pl.ds(0, K)], ssem[0]).wait()
        return 0
    lax.fori_loop(0, nblk, wdrain, 0)


def _gat_edge(xla, xr, src, dst, ea, att, we):
    mesh = plsc.VectorSubcoreMesh(core_axis_name="c", subcore_axis_name="s",
                                  num_cores=NC, num_subcores=NS)
    scratch = ([pltpu.VMEM((K,), _i32) for _ in range(NB)]
               + [pltpu.VMEM((K,), _i32) for _ in range(NB)]
               + [pltpu.VMEM((K,), _f32) for _ in range(NB)]
               + [pltpu.VMEM((K,), _i32) for _ in range(NB)]
               + [pltpu.VMEM((K, AUG), _f32) for _ in range(NB)]
               + [pltpu.VMEM((K, H), _f32) for _ in range(NB)]
               + [pltpu.VMEM((H,), _f32), pltpu.VMEM((H,), _f32)]
               + [pltpu.VMEM_SHARED((N, AUG), _f32)]
               + [pltpu.SemaphoreType.DMA for _ in range(3 * NB)])
    f = pl.kernel(
        _gat_edge_kernel,
        out_type=jax.ShapeDtypeStruct((NC, N, AUG), _f32),
        mesh=mesh,
        compiler_params=pltpu.CompilerParams(use_tc_tiling_on_sc=False),
        scratch_types=scratch,
    )
    return f(xla, xr, src, dst, ea, att, we)


def _combine_proj_body(p_ref, b_ref, wl_ref, wr_ref, x1_ref, xla_ref, xr_ref):
    p = p_ref[...]
    ps = p[0] + p[1]
    den = ps[:, H:H + 1]
    x1 = jnp.maximum(ps[:, :H] / (den + 1e-16) + b_ref[...], 0.0)
    x1_ref[...] = x1
    xl = jnp.dot(x1, wl_ref[...], preferred_element_type=_f32)
    ones = jnp.ones((BN, 1), _f32)
    zer = jnp.zeros((BN, AUG - H - 1), _f32)
    xla_ref[...] = jnp.concatenate([xl, ones, zer], axis=1)
    xr_ref[...] = jnp.dot(x1, wr_ref[...], preferred_element_type=_f32)


def _combine_proj(part, b, wl, wr):
    return pl.pallas_call(
        _combine_proj_body,
        grid=(GRID,),
        in_specs=[
            pl.BlockSpec((NC, BN, AUG), lambda i: (0, i, 0)),
            pl.BlockSpec((1, H), lambda i: (0, 0)),
            pl.BlockSpec((H, H), lambda i: (0, 0)),
            pl.BlockSpec((H, H), lambda i: (0, 0)),
        ],
        out_specs=[
            pl.BlockSpec((BN, H), lambda i: (i, 0)),
            pl.BlockSpec((BN, AUG), lambda i: (i, 0)),
            pl.BlockSpec((BN, H), lambda i: (i, 0)),
        ],
        out_shape=[
            jax.ShapeDtypeStruct((N, H), _f32),
            jax.ShapeDtypeStruct((N, AUG), _f32),
            jax.ShapeDtypeStruct((N, H), _f32),
        ],
    )(part, b, wl, wr)


def _final_body(p_ref, b_ref, x1_ref, a1w_ref, a1b_ref, a2w_ref, a2b_ref,
                c1w_ref, c1b_ref, c2w_ref, c2b_ref, ls_ref,
                mu_ref, std_ref, val_ref):
    p = p_ref[...]
    ps = p[0] + p[1]
    den = ps[:, H:H + 1]
    x2 = (jnp.maximum(ps[:, :H] / (den + 1e-16) + b_ref[...], 0.0)
          + x1_ref[...])
    ha = jnp.maximum(
        jnp.dot(x2, a1w_ref[...], preferred_element_type=_f32) + a1b_ref[...],
        0.0)
    mu_ref[...] = (jnp.dot(ha, a2w_ref[...], preferred_element_type=_f32)
                   + a2b_ref[...])
    std_ref[...] = jnp.broadcast_to(jnp.exp(ls_ref[...]), (BN, ACT))
    hv = jnp.maximum(
        jnp.dot(x2, c1w_ref[...], preferred_element_type=_f32) + c1b_ref[...],
        0.0)
    val_ref[...] = (jnp.dot(hv, c2w_ref[...], preferred_element_type=_f32)
                    + c2b_ref[...])


def _final(part, b, x1, a1w, a1b, a2w, a2b, c1w, c1b, c2w, c2b, log_std):
    return pl.pallas_call(
        _final_body,
        grid=(GRID,),
        in_specs=[
            pl.BlockSpec((NC, BN, AUG), lambda i: (0, i, 0)),
            pl.BlockSpec((1, H), lambda i: (0, 0)),
            pl.BlockSpec((BN, H), lambda i: (i, 0)),
            pl.BlockSpec((H, H), lambda i: (0, 0)),
            pl.BlockSpec((1, H), lambda i: (0, 0)),
            pl.BlockSpec((H, ACT), lambda i: (0, 0)),
            pl.BlockSpec((1, ACT), lambda i: (0, 0)),
            pl.BlockSpec((H, H), lambda i: (0, 0)),
            pl.BlockSpec((1, H), lambda i: (0, 0)),
            pl.BlockSpec((H, 1), lambda i: (0, 0)),
            pl.BlockSpec((1, 1), lambda i: (0, 0)),
            pl.BlockSpec((1, ACT), lambda i: (0, 0)),
        ],
        out_specs=[
            pl.BlockSpec((BN, ACT), lambda i: (i, 0)),
            pl.BlockSpec((BN, ACT), lambda i: (i, 0)),
            pl.BlockSpec((BN, 1), lambda i: (i, 0)),
        ],
        out_shape=[
            jax.ShapeDtypeStruct((N, ACT), _f32),
            jax.ShapeDtypeStruct((N, ACT), _f32),
            jax.ShapeDtypeStruct((N, 1), _f32),
        ],
    )(part, b, x1, a1w, a1b, a2w, a2b, c1w, c1b, c2w, c2b, log_std)


def kernel(x_seq, edge_index, edge_attr, Wih, Whh, bih, bhh, Wl1, Wr1, We1,
           att1, b1, Wl2, Wr2, We2, att2, b2, A1w, A1b, A2w, A2b, C1w, C1b,
           C2w, C2b, log_std):
    src = edge_index[0]
    dst = edge_index[1]
    ea = edge_attr[:, 0]

    xla1, xr1 = _lstm_proj(x_seq, Wih.T, Whh.T,
                           (bih + bhh).reshape(1, 4 * H), Wl1, Wr1)
    part1 = _gat_edge(xla1, xr1, src, dst, ea, att1, We1[0])
    x1, xla2, xr2 = _combine_proj(part1, b1.reshape(1, H), Wl2, Wr2)
    part2 = _gat_edge(xla2, xr2, src, dst, ea, att2, We2[0])
    mu, std, value = _final(part2, b2.reshape(1, H), x1, A1w,
                            A1b.reshape(1, H), A2w, A2b.reshape(1, ACT), C1w,
                            C1b.reshape(1, H), C2w, C2b.reshape(1, 1),
                            log_std.reshape(1, ACT))
    return (mu, std, value[:, 0])


# speedup vs baseline: 1.3379x; 1.2150x over previous
import jax
import jax.numpy as jnp
from jax import lax
from jax.experimental import pallas as pl
from jax.experimental.pallas import tpu as pltpu
from jax.experimental.pallas import tpu_sc as plsc

N = 10000
T = 8
OBS = 128
H = 128
ACT = 8
E = 320000

AUG = H + 16
BN = 1000
GRID = N // BN

NC = 2
NS = 16
NW = NC * NS
EPW = E // NW
K = 16
NCH = EPW // K
NVR = AUG // 16

_f32 = jnp.float32
_i32 = jnp.int32


def _lstm_body(x_ref, wih_ref, whh_ref, b_ref, wl_ref, wr_ref,
               xla_ref, xr_ref):
    x = x_ref[...]
    h = jnp.zeros((BN, H), _f32)
    c = jnp.zeros((BN, H), _f32)
    wih = wih_ref[...]
    whh = whh_ref[...]
    b = b_ref[...]
    for t in range(T):
        xt = x[:, t, :]
        z = (jnp.dot(xt, wih, preferred_element_type=_f32)
             + jnp.dot(h, whh, preferred_element_type=_f32) + b)
        i_ = jax.nn.sigmoid(z[:, 0:H])
        f_ = jax.nn.sigmoid(z[:, H:2 * H])
        g_ = jnp.tanh(z[:, 2 * H:3 * H])
        o_ = jax.nn.sigmoid(z[:, 3 * H:4 * H])
        c = f_ * c + i_ * g_
        h = o_ * jnp.tanh(c)
    xl = jnp.dot(h, wl_ref[...], preferred_element_type=_f32)
    ones = jnp.ones((BN, 1), _f32)
    zer = jnp.zeros((BN, AUG - H - 1), _f32)
    xla_ref[...] = jnp.concatenate([xl, ones, zer], axis=1)
    xr_ref[...] = jnp.dot(h, wr_ref[...], preferred_element_type=_f32)


def _lstm_proj(x_seq, wih_t, whh_t, bias, wl, wr):
    return pl.pallas_call(
        _lstm_body,
        grid=(GRID,),
        in_specs=[
            pl.BlockSpec((BN, T, OBS), lambda i: (i, 0, 0)),
            pl.BlockSpec((OBS, 4 * H), lambda i: (0, 0)),
            pl.BlockSpec((H, 4 * H), lambda i: (0, 0)),
            pl.BlockSpec((1, 4 * H), lambda i: (0, 0)),
            pl.BlockSpec((H, H), lambda i: (0, 0)),
            pl.BlockSpec((H, H), lambda i: (0, 0)),
        ],
        out_specs=[
            pl.BlockSpec((BN, AUG), lambda i: (i, 0)),
            pl.BlockSpec((BN, H), lambda i: (i, 0)),
        ],
        out_shape=[
            jax.ShapeDtypeStruct((N, AUG), _f32),
            jax.ShapeDtypeStruct((N, H), _f32),
        ],
    )(x_seq, wih_t, whh_t, bias, wl, wr)


def _hsum_splat(v):
    for sh in (8, 4, 2, 1):
        perm = jnp.bitwise_xor(lax.iota(_i32, 16), sh)
        v = v + jnp.take(v, perm)
    return v


NB = 5


def _gat_edge_kernel(*refs):
    (xla_hbm, xr_hbm, src_hbm, dst_hbm, ea_hbm, att_hbm, we_hbm,
     out_hbm) = refs[:8]
    i0 = 8
    srcb = list(refs[i0:i0 + NB]); i0 += NB
    dstb = list(refs[i0:i0 + NB]); i0 += NB
    eab = list(refs[i0:i0 + NB]); i0 += NB
    dstS = list(refs[i0:i0 + NB]); i0 += NB
    xlb = list(refs[i0:i0 + NB]); i0 += NB
    xrb = list(refs[i0:i0 + NB]); i0 += NB
    attbuf, webuf, num_sh = refs[i0:i0 + 3]; i0 += 3
    isem = list(refs[i0:i0 + NB]); i0 += NB
    gsem = list(refs[i0:i0 + NB]); i0 += NB
    ssem = list(refs[i0:i0 + NB]); i0 += NB

    cid = lax.axis_index("c")
    sid = lax.axis_index("s")
    wid = sid * NC + cid
    ebase = wid * EPW

    pltpu.sync_copy(att_hbm, attbuf)
    pltpu.sync_copy(we_hbm, webuf)
    att_v = [attbuf[pl.ds(j * 16, 16)] for j in range(H // 16)]
    we_v = [webuf[pl.ds(j * 16, 16)] for j in range(H // 16)]

    def fire_idx(c, b):
        base = ebase + c * K
        pltpu.async_copy(src_hbm.at[pl.ds(base, K)], srcb[b], isem[b])
        pltpu.async_copy(dst_hbm.at[pl.ds(base, K)], dstb[b], isem[b])
        pltpu.async_copy(ea_hbm.at[pl.ds(base, K)], eab[b], isem[b])

    def wait_idx(b):
        pltpu.make_async_copy(src_hbm.at[pl.ds(0, K)], srcb[b], isem[b]).wait()
        pltpu.make_async_copy(dst_hbm.at[pl.ds(0, K)], dstb[b], isem[b]).wait()
        pltpu.make_async_copy(ea_hbm.at[pl.ds(0, K)], eab[b], isem[b]).wait()

    def fire_gathers(b):
        pltpu.async_copy(xla_hbm.at[srcb[b]], xlb[b], gsem[b])
        pltpu.async_copy(xr_hbm.at[dstb[b]], xrb[b], gsem[b])

    def wait_gathers(b):
        pltpu.make_async_copy(xla_hbm.at[srcb[b]], xlb[b], gsem[b]).wait()
        pltpu.make_async_copy(xr_hbm.at[dstb[b]], xrb[b], gsem[b]).wait()

    def fire_scatter(b):
        pltpu.async_copy(xlb[b], num_sh.at[dstS[b]], ssem[b], add=True)

    def wait_scatter(b):
        pltpu.make_async_copy(xlb[b], num_sh.at[dstS[b]], ssem[b]).wait()

    for b in range(NB):
        fire_idx(b, b)

    wait_idx(0)
    fire_gathers(0)
    wait_idx(1)
    fire_gathers(1)
    wait_idx(2)
    fire_gathers(2)

    for i in range(K):
        for j in range(NVR):
            xlb[3][i, pl.ds(j * 16, 16)] = jnp.zeros((16,), _f32)
    nb_tot = N // K
    nblk = (nb_tot - 1 - sid) // NS + 1

    def zfire(t, _):
        rb = (sid + t * NS) * K
        pltpu.async_copy(xlb[3], num_sh.at[pl.ds(rb, K)], ssem[0])
        return 0
    lax.fori_loop(0, nblk, zfire, 0)

    def zdrain(t, _):
        pltpu.make_async_copy(xlb[3], num_sh.at[pl.ds(0, K)], ssem[0]).wait()
        return 0
    lax.fori_loop(0, nblk, zdrain, 0)
    plsc.subcore_barrier()

    def compute_inplace(b):
        eavec = eab[b][...]

        @plsc.parallel_loop(0, K, unroll=2)
        def _edge(e):
            eas = jnp.take(eavec, jnp.full((16,), e, _i32))
            acc0 = jnp.zeros((16,), _f32)
            acc1 = jnp.zeros((16,), _f32)
            xlv = []
            for j in range(H // 16):
                sl = pl.ds(j * 16, 16)
                xl_j = xlb[b][e, sl]
                xr_j = xrb[b][e, sl]
                xlv.append(xl_j)
                u = xl_j + xr_j + eas * we_v[j]
                t = att_v[j] * jnp.maximum(u, 0.2 * u)
                if j % 2 == 0:
                    acc0 = acc0 + t
                else:
                    acc1 = acc1 + t
            w = jnp.exp(_hsum_splat(acc0 + acc1))
            for j in range(H // 16):
                xlb[b][e, pl.ds(j * 16, 16)] = xlv[j] * w
            sl = pl.ds(H, 16)
            xlb[b][e, sl] = xlb[b][e, sl] * w

    def site(i, b):
        c = NB * i + b
        b2 = (b + 3) % NB
        wait_gathers(b)
        compute_inplace(b)
        dstS[b][...] = dstb[b][...]
        fire_scatter(b)

        @pl.when(c + NB < NCH)
        def _():
            fire_idx(c + NB, b)

        @pl.when(c + 3 < NCH)
        def _():
            @pl.when(c >= 2)
            def _():
                wait_scatter(b2)
            wait_idx(b2)
            fire_gathers(b2)

    def iter_body(i, _):
        for b in range(NB):
            site(i, b)
        return 0
    lax.fori_loop(0, NCH // NB, iter_body, 0)

    for b in range(NB):
        wait_scatter(b)
    plsc.subcore_barrier()

    def wfire(t, _):
        rb = (sid + t * NS) * K
        pltpu.async_copy(num_sh.at[pl.ds(rb, K)], out_hbm.at[cid, pl.ds(rb, K)],
                         ssem[0])
        return 0
    lax.fori_loop(0, nblk, wfire, 0)

    def wdrain(t, _):
        pltpu.make_async_copy(num_sh.at[pl.ds(0, K)],
                              out_hbm.at[cid, pl.ds(0, K)], ssem[0]).wait()
        return 0
    lax.fori_loop(0, nblk, wdrain, 0)


def _gat_edge(xla, xr, src, dst, ea, att, we):
    mesh = plsc.VectorSubcoreMesh(core_axis_name="c", subcore_axis_name="s",
                                  num_cores=NC, num_subcores=NS)
    scratch = ([pltpu.VMEM((K,), _i32) for _ in range(NB)]
               + [pltpu.VMEM((K,), _i32) for _ in range(NB)]
               + [pltpu.VMEM((K,), _f32) for _ in range(NB)]
               + [pltpu.VMEM((K,), _i32) for _ in range(NB)]
               + [pltpu.VMEM((K, AUG), _f32) for _ in range(NB)]
               + [pltpu.VMEM((K, H), _f32) for _ in range(NB)]
               + [pltpu.VMEM((H,), _f32), pltpu.VMEM((H,), _f32)]
               + [pltpu.VMEM_SHARED((N, AUG), _f32)]
               + [pltpu.SemaphoreType.DMA for _ in range(3 * NB)])
    f = pl.kernel(
        _gat_edge_kernel,
        out_type=jax.ShapeDtypeStruct((NC, N, AUG), _f32),
        mesh=mesh,
        compiler_params=pltpu.CompilerParams(use_tc_tiling_on_sc=False),
        scratch_types=scratch,
    )
    return f(xla, xr, src, dst, ea, att, we)


def _combine_proj_body(p_ref, b_ref, wl_ref, wr_ref, x1_ref, xla_ref, xr_ref):
    p = p_ref[...]
    ps = p[0] + p[1]
    den = ps[:, H:H + 1]
    x1 = jnp.maximum(ps[:, :H] / (den + 1e-16) + b_ref[...], 0.0)
    x1_ref[...] = x1
    xl = jnp.dot(x1, wl_ref[...], preferred_element_type=_f32)
    ones = jnp.ones((BN, 1), _f32)
    zer = jnp.zeros((BN, AUG - H - 1), _f32)
    xla_ref[...] = jnp.concatenate([xl, ones, zer], axis=1)
    xr_ref[...] = jnp.dot(x1, wr_ref[...], preferred_element_type=_f32)


def _combine_proj(part, b, wl, wr):
    return pl.pallas_call(
        _combine_proj_body,
        grid=(GRID,),
        in_specs=[
            pl.BlockSpec((NC, BN, AUG), lambda i: (0, i, 0)),
            pl.BlockSpec((1, H), lambda i: (0, 0)),
            pl.BlockSpec((H, H), lambda i: (0, 0)),
            pl.BlockSpec((H, H), lambda i: (0, 0)),
        ],
        out_specs=[
            pl.BlockSpec((BN, H), lambda i: (i, 0)),
            pl.BlockSpec((BN, AUG), lambda i: (i, 0)),
            pl.BlockSpec((BN, H), lambda i: (i, 0)),
        ],
        out_shape=[
            jax.ShapeDtypeStruct((N, H), _f32),
            jax.ShapeDtypeStruct((N, AUG), _f32),
            jax.ShapeDtypeStruct((N, H), _f32),
        ],
    )(part, b, wl, wr)


def _final_body(p_ref, b_ref, x1_ref, a1w_ref, a1b_ref, a2w_ref, a2b_ref,
                c1w_ref, c1b_ref, c2w_ref, c2b_ref, ls_ref,
                mu_ref, std_ref, val_ref):
    p = p_ref[...]
    ps = p[0] + p[1]
    den = ps[:, H:H + 1]
    x2 = (jnp.maximum(ps[:, :H] / (den + 1e-16) + b_ref[...], 0.0)
          + x1_ref[...])
    ha = jnp.maximum(
        jnp.dot(x2, a1w_ref[...], preferred_element_type=_f32) + a1b_ref[...],
        0.0)
    mu_ref[...] = (jnp.dot(ha, a2w_ref[...], preferred_element_type=_f32)
                   + a2b_ref[...])
    std_ref[...] = jnp.broadcast_to(jnp.exp(ls_ref[...]), (BN, ACT))
    hv = jnp.maximum(
        jnp.dot(x2, c1w_ref[...], preferred_element_type=_f32) + c1b_ref[...],
        0.0)
    val_ref[...] = (jnp.dot(hv, c2w_ref[...], preferred_element_type=_f32)
                    + c2b_ref[...])


def _final(part, b, x1, a1w, a1b, a2w, a2b, c1w, c1b, c2w, c2b, log_std):
    return pl.pallas_call(
        _final_body,
        grid=(GRID,),
        in_specs=[
            pl.BlockSpec((NC, BN, AUG), lambda i: (0, i, 0)),
            pl.BlockSpec((1, H), lambda i: (0, 0)),
            pl.BlockSpec((BN, H), lambda i: (i, 0)),
            pl.BlockSpec((H, H), lambda i: (0, 0)),
            pl.BlockSpec((1, H), lambda i: (0, 0)),
            pl.BlockSpec((H, ACT), lambda i: (0, 0)),
            pl.BlockSpec((1, ACT), lambda i: (0, 0)),
            pl.BlockSpec((H, H), lambda i: (0, 0)),
            pl.BlockSpec((1, H), lambda i: (0, 0)),
            pl.BlockSpec((H, 1), lambda i: (0, 0)),
            pl.BlockSpec((1, 1), lambda i: (0, 0)),
            pl.BlockSpec((1, ACT), lambda i: (0, 0)),
        ],
        out_specs=[
            pl.BlockSpec((BN, ACT), lambda i: (i, 0)),
            pl.BlockSpec((BN, ACT), lambda i: (i, 0)),
            pl.BlockSpec((BN, 1), lambda i: (i, 0)),
        ],
        out_shape=[
            jax.ShapeDtypeStruct((N, ACT), _f32),
            jax.ShapeDtypeStruct((N, ACT), _f32),
            jax.ShapeDtypeStruct((N, 1), _f32),
        ],
    )(part, b, x1, a1w, a1b, a2w, a2b, c1w, c1b, c2w, c2b, log_std)


def kernel(x_seq, edge_index, edge_attr, Wih, Whh, bih, bhh, Wl1, Wr1, We1,
           att1, b1, Wl2, Wr2, We2, att2, b2, A1w, A1b, A2w, A2b, C1w, C1b,
           C2w, C2b, log_std):
    src = edge_index[0]
    dst = edge_index[1]
    ea = edge_attr[:, 0]

    xla1, xr1 = _lstm_proj(x_seq, Wih.T, Whh.T,
                           (bih + bhh).reshape(1, 4 * H), Wl1, Wr1)
    part1 = _gat_edge(xla1, xr1, src, dst, ea, att1, We1[0])
    x1, xla2, xr2 = _combine_proj(part1, b1.reshape(1, H), Wl2, Wr2)
    part2 = _gat_edge(xla2, xr2, src, dst, ea, att2, We2[0])
    mu, std, value = _final(part2, b2.reshape(1, H), x1, A1w,
                            A1b.reshape(1, H), A2w, A2b.reshape(1, ACT), C1w,
                            C1b.reshape(1, H), C2w, C2b.reshape(1, 1),
                            log_std.reshape(1, ACT))
    return (mu, std, value[:, 0])
